# baseline (speedup 1.0000x reference)
import functools

import jax
import jax.numpy as jnp
from jax import lax
from jax.experimental import pallas as pl
from jax.experimental.pallas import tpu as pltpu
from jax.experimental.pallas import tpu_sc as plsc

B, T = 1024, 50
V1, D, H, O = 100001, 300, 128, 4


_V0 = 100000
_PACK = 32
_G = _V0 // _PACK
_GBLK = 256
_WIN = 512
_LANES = _PACK * D

_SA = [min(300 * k // 128 * 128, _LANES - _WIN) for k in range(_PACK)]
_OFF = [300 * k - _SA[k] for k in range(_PACK)]


def _proj_body(emb_ref, wsh_ref, bc_ref, p_ref):
    e = emb_ref[...].astype(jnp.bfloat16)
    bc = bc_ref[...]
    for k in range(_PACK):
        ek = e[:, _SA[k]:_SA[k] + _WIN]
        u = lax.dot_general(ek, wsh_ref[k], (((1,), (0,)), ((), ())),
                            preferred_element_type=jnp.float32) + bc
        pf_bits = lax.bitcast_convert_type(
            u[:, :H].astype(jnp.bfloat16).astype(jnp.float32), jnp.uint32)
        pb_bits = lax.bitcast_convert_type(
            u[:, H:].astype(jnp.bfloat16).astype(jnp.float32), jnp.uint32)
        packed = (pf_bits >> 16) | pb_bits
        p_ref[k] = lax.bitcast_convert_type(packed, jnp.int32)


def _tc_project(emb2, Wsh, bc):
    nv = pl.cdiv(_G, _GBLK)
    return pl.pallas_call(
        _proj_body,
        grid=(nv,),
        in_specs=[
            pl.BlockSpec((_GBLK, _LANES), lambda i: (i, 0)),
            pl.BlockSpec((_PACK, _WIN, 2 * H), lambda i: (0, 0, 0)),
            pl.BlockSpec((1, 2 * H), lambda i: (0, 0)),
        ],
        out_specs=pl.BlockSpec((_PACK, _GBLK, H), lambda i: (0, i, 0)),
        out_shape=jax.ShapeDtypeStruct((_PACK, _G, H), jnp.int32),
        compiler_params=pltpu.CompilerParams(
            dimension_semantics=("parallel",)),
    )(emb2, Wsh, bc)


_N = B * T
_CHUNK = 200


def _sc_gather(p, idx):
    info = plsc.get_sparse_core_info()
    nw = info.num_cores * info.num_subcores
    per_w = _N // nw
    nch = per_w // _CHUNK
    mesh = plsc.VectorSubcoreMesh(core_axis_name="c", subcore_axis_name="s")

    @functools.partial(
        pl.kernel,
        mesh=mesh,
        out_type=jax.ShapeDtypeStruct((_N, H), jnp.int32),
        scratch_types=[
            pltpu.VMEM((_CHUNK,), jnp.int32),
            pltpu.VMEM((_CHUNK, H), jnp.int32),
            pltpu.SemaphoreType.DMA,
        ],
    )
    def k(p_hbm, idx_hbm, u_hbm, idx_v, rows_v, sem):
        wid = lax.axis_index("s") * info.num_cores + lax.axis_index("c")
        base = wid * per_w
        for c in range(nch):
            off = base + c * _CHUNK
            pltpu.sync_copy(idx_hbm.at[pl.ds(off, _CHUNK)], idx_v)
            pltpu.async_copy(p_hbm.at[idx_v], rows_v, sem).wait()
            pltpu.sync_copy(rows_v, u_hbm.at[pl.ds(off, _CHUNK)])

    return k(p, idx)


BB = 512
TT = 10
NB = B // BB
NT = T // TT


def _rnn_body(u_ref, Whh_ref, Wfcf_ref, Wfcb_ref, bfc_ref, out_ref, h_ref):
    t = pl.program_id(1)

    @pl.when(t == 0)
    def _():
        h_ref[...] = jnp.zeros_like(h_ref)

    uu = lax.bitcast_convert_type(u_ref[...], jnp.uint32)
    u = lax.bitcast_convert_type(uu << 16, jnp.float32)
    h = h_ref[...]
    Whh = Whh_ref[...]
    for tt in range(TT):
        hh = lax.dot_general(h, Whh, (((1,), (1,)), ((), ())),
                             preferred_element_type=jnp.float32)
        h = jnp.tanh(u[tt] + hh)
    h_ref[...] = h

    @pl.when(t == NT - 1)
    def _():
        ub = lax.bitcast_convert_type(
            uu[TT - 1] & jnp.uint32(0xFFFF0000), jnp.float32)
        hb = jnp.tanh(ub)
        logits = (lax.dot_general(h, Wfcf_ref[...], (((1,), (1,)), ((), ())),
                                  preferred_element_type=jnp.float32)
                  + lax.dot_general(hb, Wfcb_ref[...],
                                    (((1,), (1,)), ((), ())),
                                    preferred_element_type=jnp.float32)
                  + bfc_ref[...])
        m = jnp.max(logits, axis=1, keepdims=True)
        e = jnp.exp(logits - m)
        out_ref[...] = e / jnp.sum(e, axis=1, keepdims=True)


def _tc_rnn(u_tbh, W_hh_f, Wfcf, Wfcb, bfc):
    return pl.pallas_call(
        _rnn_body,
        grid=(NB, NT),
        in_specs=[
            pl.BlockSpec((TT, BB, H), lambda b, t: (t, b, 0)),
            pl.BlockSpec((H, H), lambda b, t: (0, 0)),
            pl.BlockSpec((O, H), lambda b, t: (0, 0)),
            pl.BlockSpec((O, H), lambda b, t: (0, 0)),
            pl.BlockSpec((1, O), lambda b, t: (0, 0)),
        ],
        out_specs=pl.BlockSpec((BB, O), lambda b, t: (b, 0)),
        out_shape=jax.ShapeDtypeStruct((B, O), jnp.float32),
        scratch_shapes=[pltpu.VMEM((BB, H), jnp.float32)],
        compiler_params=pltpu.CompilerParams(
            dimension_semantics=("parallel", "arbitrary")),
    )(u_tbh, W_hh_f, Wfcf, Wfcb, bfc)


def kernel(x, emb, W_ih_f, W_hh_f, b_ih_f, b_hh_f,
           W_ih_b, W_hh_b, b_ih_b, b_hh_b, W_fc, b_fc):
    bc = jnp.concatenate([b_ih_f + b_hh_f, b_ih_b + b_hh_b]).reshape(1, 2 * H)
    Wcat = jnp.concatenate([W_ih_f.T, W_ih_b.T], axis=1)
    Wsh = jnp.stack([
        jnp.pad(Wcat, ((_OFF[k], _WIN - D - _OFF[k]), (0, 0)))
        for k in range(_PACK)
    ]).astype(jnp.bfloat16)
    emb2 = emb[:_V0].reshape(_G, _LANES)
    p = _tc_project(emb2, Wsh, bc).reshape(_V0, H)

    idx = jnp.transpose(x).reshape(-1)
    idx = (idx % _PACK) * _G + idx // _PACK
    u = _sc_gather(p, idx)
    u_tbh = u.reshape(T, B, H)

    Wfcf = W_fc[:, :H]
    Wfcb = W_fc[:, H:]
    bfc = b_fc.reshape(1, O)

    return _tc_rnn(u_tbh, W_hh_f, Wfcf, Wfcb, bfc)


# speedup vs baseline: 3.1123x; 3.1123x over previous
import functools

import jax
import jax.numpy as jnp
from jax import lax
from jax.experimental import pallas as pl
from jax.experimental.pallas import tpu as pltpu
from jax.experimental.pallas import tpu_sc as plsc

B, T = 1024, 50
V1, D, H, O = 100001, 300, 128, 4


_BV = 8192


def _proj_body(e0_ref, e1_ref, e2_ref, wc_ref, bc_ref, p_ref):
    lane = lax.broadcasted_iota(jnp.int32, (_BV, H), 1)
    e2 = jnp.where(lane < D - 2 * H, e2_ref[...], 0.0)
    u = (lax.dot_general(e0_ref[...], wc_ref[0], (((1,), (0,)), ((), ())),
                         preferred_element_type=jnp.float32)
         + lax.dot_general(e1_ref[...], wc_ref[1], (((1,), (0,)), ((), ())),
                           preferred_element_type=jnp.float32)
         + lax.dot_general(e2, wc_ref[2], (((1,), (0,)), ((), ())),
                           preferred_element_type=jnp.float32)
         + bc_ref[...])
    pf_bits = lax.bitcast_convert_type(
        u[:, :H].astype(jnp.bfloat16).astype(jnp.float32), jnp.uint32)
    pb_bits = lax.bitcast_convert_type(
        u[:, H:].astype(jnp.bfloat16).astype(jnp.float32), jnp.uint32)
    packed = (pf_bits >> 16) | pb_bits
    p_ref[...] = lax.bitcast_convert_type(packed, jnp.int32)


def _tc_project(emb, Wc, bc):
    nv = pl.cdiv(V1, _BV)
    return pl.pallas_call(
        _proj_body,
        grid=(nv,),
        in_specs=[
            pl.BlockSpec((_BV, H), lambda i: (i, 0)),
            pl.BlockSpec((_BV, H), lambda i: (i, 1)),
            pl.BlockSpec((_BV, H), lambda i: (i, 2)),
            pl.BlockSpec((3, H, 2 * H), lambda i: (0, 0, 0)),
            pl.BlockSpec((1, 2 * H), lambda i: (0, 0)),
        ],
        out_specs=pl.BlockSpec((_BV, H), lambda i: (i, 0)),
        out_shape=jax.ShapeDtypeStruct((V1, H), jnp.int32),
        compiler_params=pltpu.CompilerParams(
            dimension_semantics=("parallel",)),
    )(emb, emb, emb, Wc, bc)


_N = B * T
_CHUNK = 200


def _sc_gather(p, idx):
    info = plsc.get_sparse_core_info()
    nw = info.num_cores * info.num_subcores
    per_w = _N // nw
    nch = per_w // _CHUNK
    mesh = plsc.VectorSubcoreMesh(core_axis_name="c", subcore_axis_name="s")

    @functools.partial(
        pl.kernel,
        mesh=mesh,
        out_type=jax.ShapeDtypeStruct((_N, H), jnp.int32),
        scratch_types=[
            pltpu.VMEM((_CHUNK,), jnp.int32),
            pltpu.VMEM((_CHUNK, H), jnp.int32),
            pltpu.SemaphoreType.DMA,
        ],
    )
    def k(p_hbm, idx_hbm, u_hbm, idx_v, rows_v, sem):
        wid = lax.axis_index("s") * info.num_cores + lax.axis_index("c")
        base = wid * per_w
        for c in range(nch):
            off = base + c * _CHUNK
            pltpu.sync_copy(idx_hbm.at[pl.ds(off, _CHUNK)], idx_v)
            pltpu.async_copy(p_hbm.at[idx_v], rows_v, sem).wait()
            pltpu.sync_copy(rows_v, u_hbm.at[pl.ds(off, _CHUNK)])

    return k(p, idx)


BB = 512
TT = 10
NB = B // BB
NT = T // TT


def _rnn_body(u_ref, Whh_ref, Wfcf_ref, Wfcb_ref, bfc_ref, out_ref, h_ref):
    t = pl.program_id(1)

    @pl.when(t == 0)
    def _():
        h_ref[...] = jnp.zeros_like(h_ref)

    uu = lax.bitcast_convert_type(u_ref[...], jnp.uint32)
    u = lax.bitcast_convert_type(uu << 16, jnp.float32)
    h = h_ref[...]
    Whh = Whh_ref[...]
    for tt in range(TT):
        hh = lax.dot_general(h, Whh, (((1,), (1,)), ((), ())),
                             preferred_element_type=jnp.float32)
        h = jnp.tanh(u[tt] + hh)
    h_ref[...] = h

    @pl.when(t == NT - 1)
    def _():
        ub = lax.bitcast_convert_type(
            uu[TT - 1] & jnp.uint32(0xFFFF0000), jnp.float32)
        hb = jnp.tanh(ub)
        logits = (lax.dot_general(h, Wfcf_ref[...], (((1,), (1,)), ((), ())),
                                  preferred_element_type=jnp.float32)
                  + lax.dot_general(hb, Wfcb_ref[...],
                                    (((1,), (1,)), ((), ())),
                                    preferred_element_type=jnp.float32)
                  + bfc_ref[...])
        m = jnp.max(logits, axis=1, keepdims=True)
        e = jnp.exp(logits - m)
        out_ref[...] = e / jnp.sum(e, axis=1, keepdims=True)


def _tc_rnn(u_tbh, W_hh_f, Wfcf, Wfcb, bfc):
    return pl.pallas_call(
        _rnn_body,
        grid=(NB, NT),
        in_specs=[
            pl.BlockSpec((TT, BB, H), lambda b, t: (t, b, 0)),
            pl.BlockSpec((H, H), lambda b, t: (0, 0)),
            pl.BlockSpec((O, H), lambda b, t: (0, 0)),
            pl.BlockSpec((O, H), lambda b, t: (0, 0)),
            pl.BlockSpec((1, O), lambda b, t: (0, 0)),
        ],
        out_specs=pl.BlockSpec((BB, O), lambda b, t: (b, 0)),
        out_shape=jax.ShapeDtypeStruct((B, O), jnp.float32),
        scratch_shapes=[pltpu.VMEM((BB, H), jnp.float32)],
        compiler_params=pltpu.CompilerParams(
            dimension_semantics=("parallel", "arbitrary")),
    )(u_tbh, W_hh_f, Wfcf, Wfcb, bfc)


def kernel(x, emb, W_ih_f, W_hh_f, b_ih_f, b_hh_f,
           W_ih_b, W_hh_b, b_ih_b, b_hh_b, W_fc, b_fc):
    bc = jnp.concatenate([b_ih_f + b_hh_f, b_ih_b + b_hh_b]).reshape(1, 2 * H)
    Wcat = jnp.concatenate([W_ih_f.T, W_ih_b.T], axis=1)
    Wc = jnp.pad(Wcat, ((0, 3 * H - D), (0, 0))).reshape(3, H, 2 * H)
    p = _tc_project(emb, Wc, bc)

    idx = jnp.transpose(x).reshape(-1)
    u = _sc_gather(p, idx)
    u_tbh = u.reshape(T, B, H)

    Wfcf = W_fc[:, :H]
    Wfcb = W_fc[:, H:]
    bfc = b_fc.reshape(1, O)

    return _tc_rnn(u_tbh, W_hh_f, Wfcf, Wfcb, bfc)


# speedup vs baseline: 3.1838x; 1.0230x over previous
import functools

import jax
import jax.numpy as jnp
from jax import lax
from jax.experimental import pallas as pl
from jax.experimental.pallas import tpu as pltpu
from jax.experimental.pallas import tpu_sc as plsc

B, T = 1024, 50
V1, D, H, O = 100001, 300, 128, 4


_BV = 8192


def _proj_body(e0_ref, e1_ref, e2_ref, wc_ref, bc_ref, p_ref):
    lane = lax.broadcasted_iota(jnp.int32, (_BV, H), 1)
    e2 = jnp.where(lane < D - 2 * H, e2_ref[...], 0.0)
    u = (lax.dot_general(e0_ref[...], wc_ref[0], (((1,), (0,)), ((), ())),
                         preferred_element_type=jnp.float32)
         + lax.dot_general(e1_ref[...], wc_ref[1], (((1,), (0,)), ((), ())),
                           preferred_element_type=jnp.float32)
         + lax.dot_general(e2, wc_ref[2], (((1,), (0,)), ((), ())),
                           preferred_element_type=jnp.float32)
         + bc_ref[...])
    pf_bits = lax.bitcast_convert_type(
        u[:, :H].astype(jnp.bfloat16).astype(jnp.float32), jnp.uint32)
    pb_bits = lax.bitcast_convert_type(
        u[:, H:].astype(jnp.bfloat16).astype(jnp.float32), jnp.uint32)
    packed = (pf_bits >> 16) | pb_bits
    p_ref[...] = lax.bitcast_convert_type(packed, jnp.int32)


def _tc_project(emb, Wc, bc):
    nv = pl.cdiv(V1, _BV)
    return pl.pallas_call(
        _proj_body,
        grid=(nv,),
        in_specs=[
            pl.BlockSpec((_BV, H), lambda i: (i, 0)),
            pl.BlockSpec((_BV, H), lambda i: (i, 1)),
            pl.BlockSpec((_BV, H), lambda i: (i, 2)),
            pl.BlockSpec((3, H, 2 * H), lambda i: (0, 0, 0)),
            pl.BlockSpec((1, 2 * H), lambda i: (0, 0)),
        ],
        out_specs=pl.BlockSpec((_BV, H), lambda i: (i, 0)),
        out_shape=jax.ShapeDtypeStruct((V1, H), jnp.int32),
        compiler_params=pltpu.CompilerParams(
            dimension_semantics=("parallel",)),
    )(emb, emb, emb, Wc, bc)


_N = B * T
_CHUNK = 200


def _sc_gather(p, idx):
    info = plsc.get_sparse_core_info()
    nw = info.num_cores * info.num_subcores
    per_w = _N // nw
    nch = per_w // _CHUNK
    mesh = plsc.VectorSubcoreMesh(core_axis_name="c", subcore_axis_name="s")

    @functools.partial(
        pl.kernel,
        mesh=mesh,
        out_type=jax.ShapeDtypeStruct((_N, H), jnp.int32),
        scratch_types=[
            pltpu.VMEM((per_w,), jnp.int32),
            pltpu.VMEM((2, _CHUNK, H), jnp.int32),
            pltpu.SemaphoreType.DMA,
            pltpu.SemaphoreType.DMA,
            pltpu.SemaphoreType.DMA,
            pltpu.SemaphoreType.DMA,
        ],
    )
    def k(p_hbm, idx_hbm, u_hbm, idx_v, rows_v, sg0, sg1, so0, so1):
        wid = lax.axis_index("s") * info.num_cores + lax.axis_index("c")
        base = wid * per_w
        sg = (sg0, sg1)
        so = (so0, so1)

        def g_copy(c):
            return pltpu.make_async_copy(
                p_hbm.at[idx_v.at[pl.ds(c * _CHUNK, _CHUNK)]],
                rows_v.at[c % 2], sg[c % 2])

        def o_copy(c):
            return pltpu.make_async_copy(
                rows_v.at[c % 2],
                u_hbm.at[pl.ds(base + c * _CHUNK, _CHUNK)], so[c % 2])

        pltpu.sync_copy(idx_hbm.at[pl.ds(base, per_w)], idx_v)
        g_copy(0).start()
        for c in range(nch):
            if c + 1 < nch:
                if c >= 1:
                    o_copy(c - 1).wait()
                g_copy(c + 1).start()
            g_copy(c).wait()
            o_copy(c).start()
        o_copy(nch - 2).wait()
        o_copy(nch - 1).wait()

    return k(p, idx)


BB = 512
TT = 10
NB = B // BB
NT = T // TT


def _rnn_body(u_ref, Whh_ref, Wfcf_ref, Wfcb_ref, bfc_ref, out_ref, h_ref):
    t = pl.program_id(1)

    @pl.when(t == 0)
    def _():
        h_ref[...] = jnp.zeros_like(h_ref)

    uu = lax.bitcast_convert_type(u_ref[...], jnp.uint32)
    u = lax.bitcast_convert_type(uu << 16, jnp.float32)
    h = h_ref[...]
    Whh = Whh_ref[...]
    for tt in range(TT):
        hh = lax.dot_general(h, Whh, (((1,), (1,)), ((), ())),
                             preferred_element_type=jnp.float32)
        h = jnp.tanh(u[tt] + hh)
    h_ref[...] = h

    @pl.when(t == NT - 1)
    def _():
        ub = lax.bitcast_convert_type(
            uu[TT - 1] & jnp.uint32(0xFFFF0000), jnp.float32)
        hb = jnp.tanh(ub)
        logits = (lax.dot_general(h, Wfcf_ref[...], (((1,), (1,)), ((), ())),
                                  preferred_element_type=jnp.float32)
                  + lax.dot_general(hb, Wfcb_ref[...],
                                    (((1,), (1,)), ((), ())),
                                    preferred_element_type=jnp.float32)
                  + bfc_ref[...])
        m = jnp.max(logits, axis=1, keepdims=True)
        e = jnp.exp(logits - m)
        out_ref[...] = e / jnp.sum(e, axis=1, keepdims=True)


def _tc_rnn(u_tbh, W_hh_f, Wfcf, Wfcb, bfc):
    return pl.pallas_call(
        _rnn_body,
        grid=(NB, NT),
        in_specs=[
            pl.BlockSpec((TT, BB, H), lambda b, t: (t, b, 0)),
            pl.BlockSpec((H, H), lambda b, t: (0, 0)),
            pl.BlockSpec((O, H), lambda b, t: (0, 0)),
            pl.BlockSpec((O, H), lambda b, t: (0, 0)),
            pl.BlockSpec((1, O), lambda b, t: (0, 0)),
        ],
        out_specs=pl.BlockSpec((BB, O), lambda b, t: (b, 0)),
        out_shape=jax.ShapeDtypeStruct((B, O), jnp.float32),
        scratch_shapes=[pltpu.VMEM((BB, H), jnp.float32)],
        compiler_params=pltpu.CompilerParams(
            dimension_semantics=("parallel", "arbitrary")),
    )(u_tbh, W_hh_f, Wfcf, Wfcb, bfc)


def kernel(x, emb, W_ih_f, W_hh_f, b_ih_f, b_hh_f,
           W_ih_b, W_hh_b, b_ih_b, b_hh_b, W_fc, b_fc):
    bc = jnp.concatenate([b_ih_f + b_hh_f, b_ih_b + b_hh_b]).reshape(1, 2 * H)
    Wcat = jnp.concatenate([W_ih_f.T, W_ih_b.T], axis=1)
    Wc = jnp.pad(Wcat, ((0, 3 * H - D), (0, 0))).reshape(3, H, 2 * H)
    p = _tc_project(emb, Wc, bc)

    idx = jnp.transpose(x).reshape(-1)
    u = _sc_gather(p, idx)
    u_tbh = u.reshape(T, B, H)

    Wfcf = W_fc[:, :H]
    Wfcb = W_fc[:, H:]
    bfc = b_fc.reshape(1, O)

    return _tc_rnn(u_tbh, W_hh_f, Wfcf, Wfcb, bfc)


# speedup vs baseline: 3.2138x; 1.0094x over previous
import functools

import jax
import jax.numpy as jnp
from jax import lax
from jax.experimental import pallas as pl
from jax.experimental.pallas import tpu as pltpu
from jax.experimental.pallas import tpu_sc as plsc

B, T = 1024, 50
V1, D, H, O = 100001, 300, 128, 4


_BV = 8192


def _proj_body(e0_ref, e1_ref, e2_ref, wc_ref, bc_ref, p_ref):
    lane = lax.broadcasted_iota(jnp.int32, (_BV, H), 1)
    e2 = jnp.where(lane < D - 2 * H, e2_ref[...], 0.0)
    u = (lax.dot_general(e0_ref[...], wc_ref[0], (((1,), (0,)), ((), ())),
                         preferred_element_type=jnp.float32)
         + lax.dot_general(e1_ref[...], wc_ref[1], (((1,), (0,)), ((), ())),
                           preferred_element_type=jnp.float32)
         + lax.dot_general(e2, wc_ref[2], (((1,), (0,)), ((), ())),
                           preferred_element_type=jnp.float32)
         + bc_ref[...])
    pf_bits = lax.bitcast_convert_type(
        u[:, :H].astype(jnp.bfloat16).astype(jnp.float32), jnp.uint32)
    pb_bits = lax.bitcast_convert_type(
        u[:, H:].astype(jnp.bfloat16).astype(jnp.float32), jnp.uint32)
    packed = (pf_bits >> 16) | pb_bits
    p_ref[...] = lax.bitcast_convert_type(packed, jnp.int32)


def _tc_project(emb, Wc, bc):
    nv = pl.cdiv(V1, _BV)
    return pl.pallas_call(
        _proj_body,
        grid=(nv,),
        in_specs=[
            pl.BlockSpec((_BV, H), lambda i: (i, 0)),
            pl.BlockSpec((_BV, H), lambda i: (i, 1)),
            pl.BlockSpec((_BV, H), lambda i: (i, 2)),
            pl.BlockSpec((3, H, 2 * H), lambda i: (0, 0, 0)),
            pl.BlockSpec((1, 2 * H), lambda i: (0, 0)),
        ],
        out_specs=pl.BlockSpec((_BV, H), lambda i: (i, 0)),
        out_shape=jax.ShapeDtypeStruct((V1, H), jnp.int32),
        compiler_params=pltpu.CompilerParams(
            dimension_semantics=("parallel",)),
    )(emb, emb, emb, Wc, bc)


_N = B * T
_CHUNK = 400


def _sc_gather(p, idx):
    info = plsc.get_sparse_core_info()
    nw = info.num_cores * info.num_subcores
    per_w = _N // nw
    nch = per_w // _CHUNK
    mesh = plsc.VectorSubcoreMesh(core_axis_name="c", subcore_axis_name="s")

    @functools.partial(
        pl.kernel,
        mesh=mesh,
        out_type=jax.ShapeDtypeStruct((_N, H), jnp.int32),
        scratch_types=[
            pltpu.VMEM((per_w,), jnp.int32),
            pltpu.VMEM((2, _CHUNK, H), jnp.int32),
            pltpu.SemaphoreType.DMA,
            pltpu.SemaphoreType.DMA,
            pltpu.SemaphoreType.DMA,
            pltpu.SemaphoreType.DMA,
        ],
    )
    def k(p_hbm, idx_hbm, u_hbm, idx_v, rows_v, sg0, sg1, so0, so1):
        wid = lax.axis_index("s") * info.num_cores + lax.axis_index("c")
        base = wid * per_w
        sg = (sg0, sg1)
        so = (so0, so1)

        def g_copy(c):
            return pltpu.make_async_copy(
                p_hbm.at[idx_v.at[pl.ds(c * _CHUNK, _CHUNK)]],
                rows_v.at[c % 2], sg[c % 2])

        def o_copy(c):
            return pltpu.make_async_copy(
                rows_v.at[c % 2],
                u_hbm.at[pl.ds(base + c * _CHUNK, _CHUNK)], so[c % 2])

        pltpu.sync_copy(idx_hbm.at[pl.ds(base, per_w)], idx_v)
        g_copy(0).start()
        for c in range(nch):
            if c + 1 < nch:
                if c >= 1:
                    o_copy(c - 1).wait()
                g_copy(c + 1).start()
            g_copy(c).wait()
            o_copy(c).start()
        o_copy(nch - 2).wait()
        o_copy(nch - 1).wait()

    return k(p, idx)


BB = 512
TT = 25
NB = B // BB
NT = T // TT


def _rnn_body(u_ref, Whh_ref, Wfcf_ref, Wfcb_ref, bfc_ref, out_ref, h_ref):
    t = pl.program_id(1)

    @pl.when(t == 0)
    def _():
        h_ref[...] = jnp.zeros_like(h_ref)

    uu = lax.bitcast_convert_type(u_ref[...], jnp.uint32)
    u = lax.bitcast_convert_type(uu << 16, jnp.float32)
    h = h_ref[...]
    Whh = Whh_ref[...]
    for tt in range(TT):
        hh = lax.dot_general(h, Whh, (((1,), (1,)), ((), ())),
                             preferred_element_type=jnp.float32)
        h = jnp.tanh(u[tt] + hh)
    h_ref[...] = h

    @pl.when(t == NT - 1)
    def _():
        ub = lax.bitcast_convert_type(
            uu[TT - 1] & jnp.uint32(0xFFFF0000), jnp.float32)
        hb = jnp.tanh(ub)
        logits = (lax.dot_general(h, Wfcf_ref[...], (((1,), (1,)), ((), ())),
                                  preferred_element_type=jnp.float32)
                  + lax.dot_general(hb, Wfcb_ref[...],
                                    (((1,), (1,)), ((), ())),
                                    preferred_element_type=jnp.float32)
                  + bfc_ref[...])
        m = jnp.max(logits, axis=1, keepdims=True)
        e = jnp.exp(logits - m)
        out_ref[...] = e / jnp.sum(e, axis=1, keepdims=True)


def _tc_rnn(u_tbh, W_hh_f, Wfcf, Wfcb, bfc):
    return pl.pallas_call(
        _rnn_body,
        grid=(NB, NT),
        in_specs=[
            pl.BlockSpec((TT, BB, H), lambda b, t: (t, b, 0)),
            pl.BlockSpec((H, H), lambda b, t: (0, 0)),
            pl.BlockSpec((O, H), lambda b, t: (0, 0)),
            pl.BlockSpec((O, H), lambda b, t: (0, 0)),
            pl.BlockSpec((1, O), lambda b, t: (0, 0)),
        ],
        out_specs=pl.BlockSpec((BB, O), lambda b, t: (b, 0)),
        out_shape=jax.ShapeDtypeStruct((B, O), jnp.float32),
        scratch_shapes=[pltpu.VMEM((BB, H), jnp.float32)],
        compiler_params=pltpu.CompilerParams(
            dimension_semantics=("parallel", "arbitrary")),
    )(u_tbh, W_hh_f, Wfcf, Wfcb, bfc)


def kernel(x, emb, W_ih_f, W_hh_f, b_ih_f, b_hh_f,
           W_ih_b, W_hh_b, b_ih_b, b_hh_b, W_fc, b_fc):
    bc = jnp.concatenate([b_ih_f + b_hh_f, b_ih_b + b_hh_b]).reshape(1, 2 * H)
    Wcat = jnp.concatenate([W_ih_f.T, W_ih_b.T], axis=1)
    Wc = jnp.pad(Wcat, ((0, 3 * H - D), (0, 0))).reshape(3, H, 2 * H)
    p = _tc_project(emb, Wc, bc)

    idx = jnp.transpose(x).reshape(-1)
    u = _sc_gather(p, idx)
    u_tbh = u.reshape(T, B, H)

    Wfcf = W_fc[:, :H]
    Wfcb = W_fc[:, H:]
    bfc = b_fc.reshape(1, O)

    return _tc_rnn(u_tbh, W_hh_f, Wfcf, Wfcb, bfc)


# speedup vs baseline: 3.2200x; 1.0019x over previous
import functools

import jax
import jax.numpy as jnp
from jax import lax
from jax.experimental import pallas as pl
from jax.experimental.pallas import tpu as pltpu
from jax.experimental.pallas import tpu_sc as plsc

B, T = 1024, 50
V1, D, H, O = 100001, 300, 128, 4


_BV = 12288


def _proj_body(e0_ref, e1_ref, e2_ref, wc_ref, bc_ref, p_ref):
    lane = lax.broadcasted_iota(jnp.int32, (_BV, H), 1)
    e2 = jnp.where(lane < D - 2 * H, e2_ref[...], 0.0)
    u = (lax.dot_general(e0_ref[...], wc_ref[0], (((1,), (0,)), ((), ())),
                         preferred_element_type=jnp.float32)
         + lax.dot_general(e1_ref[...], wc_ref[1], (((1,), (0,)), ((), ())),
                           preferred_element_type=jnp.float32)
         + lax.dot_general(e2, wc_ref[2], (((1,), (0,)), ((), ())),
                           preferred_element_type=jnp.float32)
         + bc_ref[...])
    pf_bits = lax.bitcast_convert_type(
        u[:, :H].astype(jnp.bfloat16).astype(jnp.float32), jnp.uint32)
    pb_bits = lax.bitcast_convert_type(
        u[:, H:].astype(jnp.bfloat16).astype(jnp.float32), jnp.uint32)
    packed = (pf_bits >> 16) | pb_bits
    p_ref[...] = lax.bitcast_convert_type(packed, jnp.int32)


def _tc_project(emb, Wc, bc):
    nv = pl.cdiv(V1, _BV)
    return pl.pallas_call(
        _proj_body,
        grid=(nv,),
        in_specs=[
            pl.BlockSpec((_BV, H), lambda i: (i, 0)),
            pl.BlockSpec((_BV, H), lambda i: (i, 1)),
            pl.BlockSpec((_BV, H), lambda i: (i, 2)),
            pl.BlockSpec((3, H, 2 * H), lambda i: (0, 0, 0)),
            pl.BlockSpec((1, 2 * H), lambda i: (0, 0)),
        ],
        out_specs=pl.BlockSpec((_BV, H), lambda i: (i, 0)),
        out_shape=jax.ShapeDtypeStruct((V1, H), jnp.int32),
        compiler_params=pltpu.CompilerParams(
            dimension_semantics=("parallel",)),
    )(emb, emb, emb, Wc, bc)


_N = B * T
_CHUNK = 400


def _sc_gather(p, idx):
    info = plsc.get_sparse_core_info()
    nw = info.num_cores * info.num_subcores
    per_w = _N // nw
    nch = per_w // _CHUNK
    mesh = plsc.VectorSubcoreMesh(core_axis_name="c", subcore_axis_name="s")

    @functools.partial(
        pl.kernel,
        mesh=mesh,
        out_type=jax.ShapeDtypeStruct((_N, H), jnp.int32),
        scratch_types=[
            pltpu.VMEM((per_w,), jnp.int32),
            pltpu.VMEM((2, _CHUNK, H), jnp.int32),
            pltpu.SemaphoreType.DMA,
            pltpu.SemaphoreType.DMA,
            pltpu.SemaphoreType.DMA,
            pltpu.SemaphoreType.DMA,
        ],
    )
    def k(p_hbm, idx_hbm, u_hbm, idx_v, rows_v, sg0, sg1, so0, so1):
        wid = lax.axis_index("s") * info.num_cores + lax.axis_index("c")
        base = wid * per_w
        sg = (sg0, sg1)
        so = (so0, so1)

        def g_copy(c):
            return pltpu.make_async_copy(
                p_hbm.at[idx_v.at[pl.ds(c * _CHUNK, _CHUNK)]],
                rows_v.at[c % 2], sg[c % 2])

        def o_copy(c):
            return pltpu.make_async_copy(
                rows_v.at[c % 2],
                u_hbm.at[pl.ds(base + c * _CHUNK, _CHUNK)], so[c % 2])

        pltpu.sync_copy(idx_hbm.at[pl.ds(base, per_w)], idx_v)
        g_copy(0).start()
        for c in range(nch):
            if c + 1 < nch:
                if c >= 1:
                    o_copy(c - 1).wait()
                g_copy(c + 1).start()
            g_copy(c).wait()
            o_copy(c).start()
        o_copy(nch - 2).wait()
        o_copy(nch - 1).wait()

    return k(p, idx)


BB = 512
TT = 25
NB = B // BB
NT = T // TT


def _rnn_body(u_ref, Whh_ref, Wfcf_ref, Wfcb_ref, bfc_ref, out_ref, h_ref):
    t = pl.program_id(1)

    @pl.when(t == 0)
    def _():
        h_ref[...] = jnp.zeros_like(h_ref)

    uu = lax.bitcast_convert_type(u_ref[...], jnp.uint32)
    u = lax.bitcast_convert_type(uu << 16, jnp.float32)
    h = h_ref[...]
    Whh = Whh_ref[...]
    for tt in range(TT):
        hh = lax.dot_general(h, Whh, (((1,), (1,)), ((), ())),
                             preferred_element_type=jnp.float32)
        h = jnp.tanh(u[tt] + hh)
    h_ref[...] = h

    @pl.when(t == NT - 1)
    def _():
        ub = lax.bitcast_convert_type(
            uu[TT - 1] & jnp.uint32(0xFFFF0000), jnp.float32)
        hb = jnp.tanh(ub)
        logits = (lax.dot_general(h, Wfcf_ref[...], (((1,), (1,)), ((), ())),
                                  preferred_element_type=jnp.float32)
                  + lax.dot_general(hb, Wfcb_ref[...],
                                    (((1,), (1,)), ((), ())),
                                    preferred_element_type=jnp.float32)
                  + bfc_ref[...])
        m = jnp.max(logits, axis=1, keepdims=True)
        e = jnp.exp(logits - m)
        out_ref[...] = e / jnp.sum(e, axis=1, keepdims=True)


def _tc_rnn(u_tbh, W_hh_f, Wfcf, Wfcb, bfc):
    return pl.pallas_call(
        _rnn_body,
        grid=(NB, NT),
        in_specs=[
            pl.BlockSpec((TT, BB, H), lambda b, t: (t, b, 0)),
            pl.BlockSpec((H, H), lambda b, t: (0, 0)),
            pl.BlockSpec((O, H), lambda b, t: (0, 0)),
            pl.BlockSpec((O, H), lambda b, t: (0, 0)),
            pl.BlockSpec((1, O), lambda b, t: (0, 0)),
        ],
        out_specs=pl.BlockSpec((BB, O), lambda b, t: (b, 0)),
        out_shape=jax.ShapeDtypeStruct((B, O), jnp.float32),
        scratch_shapes=[pltpu.VMEM((BB, H), jnp.float32)],
        compiler_params=pltpu.CompilerParams(
            dimension_semantics=("parallel", "arbitrary")),
    )(u_tbh, W_hh_f, Wfcf, Wfcb, bfc)


def kernel(x, emb, W_ih_f, W_hh_f, b_ih_f, b_hh_f,
           W_ih_b, W_hh_b, b_ih_b, b_hh_b, W_fc, b_fc):
    bc = jnp.concatenate([b_ih_f + b_hh_f, b_ih_b + b_hh_b]).reshape(1, 2 * H)
    Wcat = jnp.concatenate([W_ih_f.T, W_ih_b.T], axis=1)
    Wc = jnp.pad(Wcat, ((0, 3 * H - D), (0, 0))).reshape(3, H, 2 * H)
    p = _tc_project(emb, Wc, bc)

    idx = jnp.transpose(x).reshape(-1)
    u = _sc_gather(p, idx)
    u_tbh = u.reshape(T, B, H)

    Wfcf = W_fc[:, :H]
    Wfcb = W_fc[:, H:]
    bfc = b_fc.reshape(1, O)

    return _tc_rnn(u_tbh, W_hh_f, Wfcf, Wfcb, bfc)
